# Initial kernel scaffold; baseline (speedup 1.0000x reference)
import jax
import jax.numpy as jnp
from jax.experimental import pallas as pl


def kernel(weight, word_idx, topn):
    raise NotImplementedError("write your pallas kernel here")


# speedup vs baseline: 1.9861x; 1.9861x over previous
import functools

import jax
import jax.numpy as jnp
from jax import lax
from jax.experimental import pallas as pl
from jax.experimental.pallas import tpu as pltpu
from jax.experimental.pallas import tpu_sc as plsc

V = 100000
D = 300
DP = 384
Q = 1024
K = 10

G = 128
VB = 512
NB = (V + VB - 1) // VB
VPAD = NB * VB
NG = VPAD // G
GPB = VB // G

NC = 2
NS = 16
NW = NC * NS
RW = Q // NW

@functools.cache
def _sc_mesh():
    return plsc.VectorSubcoreMesh(core_axis_name="c", subcore_axis_name="s")


def _worker_id():
    return lax.axis_index("s") * NC + lax.axis_index("c")


def _k0_body(weight_hbm, idx_hbm, out_hbm, idx_v, rows_v, sem):
    base = _worker_id() * RW
    pltpu.sync_copy(idx_hbm.at[pl.ds(base, RW)], idx_v)
    pltpu.async_copy(weight_hbm.at[idx_v], rows_v, sem).wait()
    pltpu.sync_copy(rows_v, out_hbm.at[pl.ds(base, RW)])


@jax.jit
def _k0(weight, word_idx):
    return pl.kernel(
        _k0_body,
        mesh=_sc_mesh(),
        out_type=jax.ShapeDtypeStruct((Q, DP), jnp.float32),
        scratch_types=[
            pltpu.VMEM((RW,), jnp.int32),
            pltpu.VMEM((RW, DP), jnp.float32),
            pltpu.SemaphoreType.DMA,
        ],
    )(weight, word_idx)


def _k1_body(q_ref, w_ref, scores_ref, gmax_ref):
    i = pl.program_id(0)
    s = lax.dot_general(q_ref[...], w_ref[...], (((1,), (1,)), ((), ())),
                        preferred_element_type=jnp.float32)
    col = i * VB + lax.broadcasted_iota(jnp.int32, (Q, VB), 1)
    s = jnp.where(col < V, s, -jnp.inf)
    scores_ref[...] = s
    gmax_ref[...] = jnp.max(s.reshape(Q, GPB, G), axis=-1)[None]


@jax.jit
def _k1(q, weight):
    return pl.pallas_call(
        _k1_body,
        grid=(NB,),
        in_specs=[
            pl.BlockSpec((Q, DP), lambda i: (0, 0)),
            pl.BlockSpec((VB, DP), lambda i: (i, 0)),
        ],
        out_specs=[
            pl.BlockSpec((Q, VB), lambda i: (0, i)),
            pl.BlockSpec((1, Q, GPB), lambda i: (i, 0, 0)),
        ],
        out_shape=[
            jax.ShapeDtypeStruct((Q, VPAD), jnp.float32),
            jax.ShapeDtypeStruct((NB, Q, GPB), jnp.float32),
        ],
        compiler_params=pltpu.CompilerParams(
            dimension_semantics=("arbitrary",)),
    )(q, weight)


def _k2_body(gm_ref, bid_ref):
    gm = gm_ref[...]
    gio = lax.broadcasted_iota(jnp.int32, (NG, Q), 0)
    rows = []
    for _ in range(K):
        m = jnp.max(gm, axis=0, keepdims=True)
        am = jnp.min(jnp.where(gm == m, gio, NG), axis=0, keepdims=True)
        rows.append(am)
        gm = jnp.where(gio == am, -jnp.inf, gm)
    bid_ref[...] = jnp.concatenate(rows, axis=0)


@jax.jit
def _k2(gm):
    return pl.pallas_call(
        _k2_body,
        out_shape=jax.ShapeDtypeStruct((K, Q), jnp.int32),
    )(gm)


_GCHUNK = 80


def _k3a_body(scores2_hbm, seg_hbm, cand2_hbm, segidx, gath, sem):
    base = _worker_id() * (RW * K)
    pltpu.sync_copy(seg_hbm.at[pl.ds(base, RW * K)], segidx)
    copies = []
    for t in range(0, RW * K, _GCHUNK):
        copies.append(pltpu.async_copy(
            scores2_hbm.at[segidx.at[pl.ds(t, _GCHUNK)]],
            gath.at[pl.ds(t, _GCHUNK)], sem))
    for cp in copies:
        cp.wait()
    pltpu.sync_copy(gath, cand2_hbm.at[pl.ds(base, RW * K)])


@jax.jit
def _k3a(scores2, seg):
    return pl.kernel(
        _k3a_body,
        mesh=_sc_mesh(),
        out_type=jax.ShapeDtypeStruct((Q * K, G), jnp.float32),
        scratch_types=[
            pltpu.VMEM((RW * K,), jnp.int32),
            pltpu.VMEM((RW * K, G), jnp.float32),
            pltpu.SemaphoreType.DMA,
        ],
    )(scores2, seg)


def _k3b_body(cand_ref, bidt_ref, vals_ref, idxs_ref):
    s = cand_ref[...]
    bidt = bidt_ref[...]
    coff = lax.broadcasted_iota(jnp.int32, (Q, K * G), 1)
    cg = coff // G
    gb = jnp.zeros((Q, K * G), jnp.int32)
    for j in range(K):
        gb = gb + jnp.where(cg == j, bidt[:, j:j + 1], 0)
    gcol = gb * G + (coff % G)
    big = jnp.int32(1 << 30)
    vcols, icols = [], []
    for _ in range(K):
        m = jnp.max(s, axis=1, keepdims=True)
        gi = jnp.min(jnp.where(s == m, gcol, big), axis=1, keepdims=True)
        vcols.append(m)
        icols.append(gi)
        s = jnp.where(gcol == gi, -jnp.inf, s)
    vals_ref[...] = jnp.concatenate(vcols, axis=1)
    idxs_ref[...] = jnp.concatenate(icols, axis=1)


@jax.jit
def _k3b(cand, bidt):
    return pl.pallas_call(
        _k3b_body,
        out_shape=[
            jax.ShapeDtypeStruct((Q, K), jnp.float32),
            jax.ShapeDtypeStruct((Q, K), jnp.int32),
        ],
    )(cand, bidt)


def kernel(weight, word_idx, topn):
    del topn
    word_idx = word_idx.astype(jnp.int32)
    wpad = jnp.pad(weight, ((0, 0), (0, DP - D)))
    q = _k0(wpad, word_idx)
    scores, gmax = _k1(q, wpad)
    gm = gmax.transpose(0, 2, 1).reshape(NG, Q)
    bid = _k2(gm)
    seg = (jnp.arange(Q, dtype=jnp.int32)[:, None] * NG
           + bid.T.astype(jnp.int32)).reshape(-1)
    cand2 = _k3a(scores.reshape(Q * NG, G), seg)
    vals, idxs = _k3b(cand2.reshape(Q, K * G), bid.T)
    return vals, idxs


# speedup vs baseline: 3.3858x; 1.7047x over previous
import functools

import jax
import jax.numpy as jnp
from jax import lax
from jax.experimental import pallas as pl
from jax.experimental.pallas import tpu as pltpu
from jax.experimental.pallas import tpu_sc as plsc

V = 100000
D = 300
DP = 384
Q = 1024
K = 10

G = 128
VB = 512
NB = (V + VB - 1) // VB
VPAD = NB * VB
NG = VPAD // G
GPB = VB // G

NC = 2
NS = 16
NW = NC * NS
RW = Q // NW

@functools.cache
def _sc_mesh():
    return plsc.VectorSubcoreMesh(core_axis_name="c", subcore_axis_name="s")


def _worker_id():
    return lax.axis_index("s") * NC + lax.axis_index("c")


PB = 1000


def _pad_body(w_ref, o_ref):
    o_ref[:, :D] = w_ref[...]
    o_ref[:, D:] = jnp.zeros((PB, DP - D), jnp.float32)


@jax.jit
def _pad(weight):
    return pl.pallas_call(
        _pad_body,
        grid=(V // PB,),
        in_specs=[pl.BlockSpec((PB, D), lambda i: (i, 0))],
        out_specs=pl.BlockSpec((PB, DP), lambda i: (i, 0)),
        out_shape=jax.ShapeDtypeStruct((V, DP), jnp.float32),
        compiler_params=pltpu.CompilerParams(
            dimension_semantics=("arbitrary",)),
    )(weight)


def _k0_body(weight_hbm, idx_hbm, out_hbm, idx_v, rows_v, sem):
    base = _worker_id() * RW
    pltpu.sync_copy(idx_hbm.at[pl.ds(base, RW)], idx_v)
    pltpu.async_copy(weight_hbm.at[idx_v], rows_v, sem).wait()
    pltpu.sync_copy(rows_v, out_hbm.at[pl.ds(base, RW)])


@jax.jit
def _k0(weight, word_idx):
    return pl.kernel(
        _k0_body,
        mesh=_sc_mesh(),
        out_type=jax.ShapeDtypeStruct((Q, DP), jnp.float32),
        scratch_types=[
            pltpu.VMEM((RW,), jnp.int32),
            pltpu.VMEM((RW, DP), jnp.float32),
            pltpu.SemaphoreType.DMA,
        ],
    )(weight, word_idx)


def _k1_body(q_ref, w_ref, scores_ref, gmax_ref):
    i = pl.program_id(0)
    s = lax.dot_general(q_ref[...], w_ref[...], (((1,), (1,)), ((), ())),
                        preferred_element_type=jnp.float32)

    def emit(sv):
        for j in range(GPB):
            scores_ref[j, :, :] = sv[:, j * G:(j + 1) * G]
        gmax_ref[...] = jnp.max(sv.reshape(Q, GPB, G), axis=-1)[None]

    @pl.when(i < NB - 1)
    def _():
        emit(s)

    @pl.when(i == NB - 1)
    def _():
        col = i * VB + lax.broadcasted_iota(jnp.int32, (Q, VB), 1)
        emit(jnp.where(col < V, s, -jnp.inf))


@jax.jit
def _k1(q, weight):
    return pl.pallas_call(
        _k1_body,
        grid=(NB,),
        in_specs=[
            pl.BlockSpec((Q, DP), lambda i: (0, 0)),
            pl.BlockSpec((VB, DP), lambda i: (i, 0)),
        ],
        out_specs=[
            pl.BlockSpec((GPB, Q, G), lambda i: (i, 0, 0)),
            pl.BlockSpec((1, Q, GPB), lambda i: (i, 0, 0)),
        ],
        out_shape=[
            jax.ShapeDtypeStruct((NG, Q, G), jnp.float32),
            jax.ShapeDtypeStruct((NB, Q, GPB), jnp.float32),
        ],
        compiler_params=pltpu.CompilerParams(
            dimension_semantics=("arbitrary",)),
    )(q, weight)


def _k2_body(gm_ref, bid_ref):
    gm = gm_ref[...]
    gio = lax.broadcasted_iota(jnp.int32, (NG, Q), 0)
    rows = []
    for _ in range(K):
        m = jnp.max(gm, axis=0, keepdims=True)
        am = jnp.min(jnp.where(gm == m, gio, NG), axis=0, keepdims=True)
        rows.append(am)
        gm = jnp.where(gio == am, -jnp.inf, gm)
    bid_ref[...] = jnp.concatenate(rows, axis=0)


@jax.jit
def _k2(gm):
    return pl.pallas_call(
        _k2_body,
        out_shape=jax.ShapeDtypeStruct((K, Q), jnp.int32),
    )(gm)


_GCHUNK = 80


def _k3a_body(scores2_hbm, seg_hbm, cand2_hbm, segidx, gath, sem):
    base = _worker_id() * (RW * K)
    pltpu.sync_copy(seg_hbm.at[pl.ds(base, RW * K)], segidx)
    copies = []
    for t in range(0, RW * K, _GCHUNK):
        copies.append(pltpu.async_copy(
            scores2_hbm.at[segidx.at[pl.ds(t, _GCHUNK)]],
            gath.at[pl.ds(t, _GCHUNK)], sem))
    for cp in copies:
        cp.wait()
    pltpu.sync_copy(gath, cand2_hbm.at[pl.ds(base, RW * K)])


@jax.jit
def _k3a(scores2, seg):
    return pl.kernel(
        _k3a_body,
        mesh=_sc_mesh(),
        out_type=jax.ShapeDtypeStruct((Q * K, G), jnp.float32),
        scratch_types=[
            pltpu.VMEM((RW * K,), jnp.int32),
            pltpu.VMEM((RW * K, G), jnp.float32),
            pltpu.SemaphoreType.DMA,
        ],
    )(scores2, seg)


def _k3b_body(cand_ref, bidt_ref, vals_ref, idxs_ref):
    s = cand_ref[...]
    bidt = bidt_ref[...]
    coff = lax.broadcasted_iota(jnp.int32, (Q, K * G), 1)
    cg = coff // G
    gb = jnp.zeros((Q, K * G), jnp.int32)
    for j in range(K):
        gb = gb + jnp.where(cg == j, bidt[:, j:j + 1], 0)
    gcol = gb * G + (coff % G)
    big = jnp.int32(1 << 30)
    vcols, icols = [], []
    for _ in range(K):
        m = jnp.max(s, axis=1, keepdims=True)
        gi = jnp.min(jnp.where(s == m, gcol, big), axis=1, keepdims=True)
        vcols.append(m)
        icols.append(gi)
        s = jnp.where(gcol == gi, -jnp.inf, s)
    vals_ref[...] = jnp.concatenate(vcols, axis=1)
    idxs_ref[...] = jnp.concatenate(icols, axis=1)


@jax.jit
def _k3b(cand, bidt):
    return pl.pallas_call(
        _k3b_body,
        out_shape=[
            jax.ShapeDtypeStruct((Q, K), jnp.float32),
            jax.ShapeDtypeStruct((Q, K), jnp.int32),
        ],
    )(cand, bidt)


def kernel(weight, word_idx, topn):
    del topn
    word_idx = word_idx.astype(jnp.int32)
    wpad = _pad(weight)
    q = _k0(wpad, word_idx)
    scores_g, gmax = _k1(q, wpad)
    gm = gmax.transpose(0, 2, 1).reshape(NG, Q)
    bid = _k2(gm)
    seg = (bid.T.astype(jnp.int32) * Q
           + jnp.arange(Q, dtype=jnp.int32)[:, None]).reshape(-1)
    cand2 = _k3a(scores_g.reshape(NG * Q, G), seg)
    vals, idxs = _k3b(cand2.reshape(Q, K * G), bid.T)
    return vals, idxs
